# Initial kernel scaffold; baseline (speedup 1.0000x reference)
import jax
import jax.numpy as jnp
from jax.experimental import pallas as pl


def kernel(x, edge_index, edge_attr, edge_gate_type, batch, global_features, params):
    raise NotImplementedError("write your pallas kernel here")


# speedup vs baseline: 2.1732x; 2.1732x over previous
import functools

import jax
import jax.numpy as jnp
from jax import lax
from jax.experimental import pallas as pl
from jax.experimental.pallas import tpu as pltpu
from jax.experimental.pallas import tpu_sc as plsc

_N = 50000
_E = 800000
_B = 64
_NF = 128
_EF = 4
_GF = 52
_H = 64
_L = 4
_NC = 9
_NG = 8
_GE = 16

_CH = 128
_EPAD = 819200
_ROWS = _EPAD // _CH
_RPT = _ROWS // 16
_RPT_DEG = _ROWS // 32
_AGG_ROWS = 50048
_ZSL = _AGG_ROWS // 16
_NPOOL = 50176
_PPT = _NPOOL // 32
_PHALF = _PPT // 2


def _embed_body(x_ref, w_ref, b_ref, g_ref, bb_ref, o_ref):
    h = jnp.maximum(jnp.dot(x_ref[...], w_ref[...],
                            preferred_element_type=jnp.float32) + b_ref[...], 0.0)
    mu = jnp.mean(h, axis=-1, keepdims=True)
    var = jnp.mean((h - mu) ** 2, axis=-1, keepdims=True)
    o_ref[...] = (h - mu) * lax.rsqrt(var + 1e-5) * g_ref[...] + bb_ref[...]


def _embed(x, w, b, g, bb):
    grid = _N // 1000
    return pl.pallas_call(
        _embed_body,
        grid=(grid,),
        in_specs=[
            pl.BlockSpec((1000, _NF), lambda i: (i, 0)),
            pl.BlockSpec((_NF, _H), lambda i: (0, 0)),
            pl.BlockSpec((1, _H), lambda i: (0, 0)),
            pl.BlockSpec((1, _H), lambda i: (0, 0)),
            pl.BlockSpec((1, _H), lambda i: (0, 0)),
        ],
        out_specs=pl.BlockSpec((1000, _H), lambda i: (i, 0)),
        out_shape=jax.ShapeDtypeStruct((_N, _H), jnp.float32),
    )(x, w, b, g, bb)


def _ec_body(ea_ref, gt_ref, we_ref, gemb_ref, wg_ref, bm_ref, o_ref):
    gp = jnp.concatenate(
        [jnp.dot(gemb_ref[l], wg_ref[l], preferred_element_type=jnp.float32)
         for l in range(_L)], axis=-1)
    oh = (gt_ref[...] == jnp.arange(_NG, dtype=jnp.int32)[None, :]
          .astype(jnp.float32)).astype(jnp.float32)
    ec = (jnp.dot(ea_ref[...], we_ref[...], preferred_element_type=jnp.float32)
          + jnp.dot(oh, gp, preferred_element_type=jnp.float32) + bm_ref[...])
    for p in range(2 * _L):
        o_ref[p] = ec[:, p * 32:(p + 1) * 32]


def _ec_all(ea, gt, we_s, gemb_s, wg_s, bm_s):
    grid = _EPAD // 1024
    return pl.pallas_call(
        _ec_body,
        grid=(grid,),
        in_specs=[
            pl.BlockSpec((1024, _EF), lambda i: (i, 0)),
            pl.BlockSpec((1024, 1), lambda i: (i, 0)),
            pl.BlockSpec((_EF, 2 * _L * 32), lambda i: (0, 0)),
            pl.BlockSpec((_L, _NG, _GE), lambda i: (0, 0, 0)),
            pl.BlockSpec((_L, _GE, _H), lambda i: (0, 0, 0)),
            pl.BlockSpec((1, 2 * _L * 32), lambda i: (0, 0)),
        ],
        out_specs=pl.BlockSpec((2 * _L, 1024, 32), lambda i: (0, i, 0)),
        out_shape=jax.ShapeDtypeStruct((2 * _L, _EPAD, 32), jnp.float32),
    )(ea, gt, we_s, gemb_s, wg_s, bm_s)


def _z_body(h_ref, w_ref, o_ref):
    z = jnp.dot(h_ref[...], w_ref[...], preferred_element_type=jnp.float32)
    o_ref[0] = z[:, :32]
    o_ref[1] = z[:, 32:]


def _z_halves(h, w):
    grid = _N // 1000
    return pl.pallas_call(
        _z_body,
        grid=(grid,),
        in_specs=[
            pl.BlockSpec((1000, _H), lambda i: (i, 0)),
            pl.BlockSpec((_H, _H), lambda i: (0, 0)),
        ],
        out_specs=pl.BlockSpec((2, 1000, 32), lambda i: (0, i, 0)),
        out_shape=jax.ShapeDtypeStruct((2, _N, 32), jnp.float32),
    )(h, w)


def _upd_body(h_ref, agg_ref, deg_ref, w1_ref, w2a_ref, w2b_ref, b_ref, o_ref):
    deg = deg_ref[0, :, :1] + deg_ref[1, :, :1]
    inv = 1.0 / jnp.maximum(deg, 1.0)
    a0 = agg_ref[0] * inv
    a1 = agg_ref[1] * inv
    u = (jnp.dot(h_ref[...], w1_ref[...], preferred_element_type=jnp.float32)
         + jnp.dot(a0, w2a_ref[...], preferred_element_type=jnp.float32)
         + jnp.dot(a1, w2b_ref[...], preferred_element_type=jnp.float32)
         + b_ref[...])
    o_ref[...] = h_ref[...] + jnp.maximum(u, 0.0)


def _update(h, agg, degp, wu, bu):
    grid = _N // 1000
    return pl.pallas_call(
        _upd_body,
        grid=(grid,),
        in_specs=[
            pl.BlockSpec((1000, _H), lambda i: (i, 0)),
            pl.BlockSpec((2, 1000, 32), lambda i: (0, i, 0)),
            pl.BlockSpec((2, 1000, 16), lambda i: (0, i, 0)),
            pl.BlockSpec((_H, _H), lambda i: (0, 0)),
            pl.BlockSpec((32, _H), lambda i: (0, 0)),
            pl.BlockSpec((32, _H), lambda i: (0, 0)),
            pl.BlockSpec((1, _H), lambda i: (0, 0)),
        ],
        out_specs=pl.BlockSpec((1000, _H), lambda i: (i, 0)),
        out_shape=jax.ShapeDtypeStruct((_N, _H), jnp.float32),
    )(h, agg, degp, wu[:_H], wu[_H:_H + 32], wu[_H + 32:], bu)


def _psum_body(h_ref, b_ref, o_ref):
    i = pl.program_id(0)

    @pl.when(i == 0)
    def _():
        o_ref[...] = jnp.zeros_like(o_ref)

    p = (b_ref[...] == jnp.arange(_B, dtype=jnp.int32)[None, :]
         .astype(jnp.float32)).astype(jnp.float32)
    h1 = jnp.concatenate(
        [h_ref[...], jnp.ones((1000, 1), jnp.float32)], axis=1)
    o_ref[...] += lax.dot_general(p, h1, (((0,), (0,)), ((), ())),
                                  preferred_element_type=jnp.float32)


def _pool_sum(h, batchf):
    grid = _N // 1000
    return pl.pallas_call(
        _psum_body,
        grid=(grid,),
        in_specs=[
            pl.BlockSpec((1000, _H), lambda i: (i, 0)),
            pl.BlockSpec((1000, 1), lambda i: (i, 0)),
        ],
        out_specs=pl.BlockSpec((_B, _H + 1), lambda i: (0, 0)),
        out_shape=jax.ShapeDtypeStruct((_B, _H + 1), jnp.float32),
    )(h, batchf)


def _ln(v, g, b):
    mu = jnp.mean(v, axis=-1, keepdims=True)
    var = jnp.mean((v - mu) ** 2, axis=-1, keepdims=True)
    return (v - mu) * lax.rsqrt(var + 1e-5) * g + b


def _head_body(sums_ref, maxp_ref, gf_ref, gpw_ref, gpb_ref, gpg_ref, gpbb_ref,
               w1_ref, b1_ref, lng_ref, lnb_ref, w2_ref, b2_ref, wt_ref,
               bt_ref, o_ref):
    counts = sums_ref[:, _H:_H + 1]
    h_sum = sums_ref[:, :_H]
    h_mean = h_sum / jnp.maximum(counts, 1.0)
    h_max = jnp.maximum(maxp_ref[0], maxp_ref[1])
    h_max = jnp.where(counts > 0.0, h_max, 0.0)
    g = jnp.maximum(jnp.dot(gf_ref[...], gpw_ref[...],
                            preferred_element_type=jnp.float32) + gpb_ref[...], 0.0)
    g = _ln(g, gpg_ref[...], gpbb_ref[...])
    c = jnp.concatenate([h_mean, h_max, h_sum, g], axis=-1)
    c = jnp.maximum(jnp.dot(c, w1_ref[...],
                            preferred_element_type=jnp.float32) + b1_ref[...], 0.0)
    c = _ln(c, lng_ref[...], lnb_ref[...])
    c = jnp.maximum(jnp.dot(c, w2_ref[...],
                            preferred_element_type=jnp.float32) + b2_ref[...], 0.0)
    o_ref[...] = jnp.dot(c, wt_ref[...],
                         preferred_element_type=jnp.float32) + bt_ref[...]


def _head(sums, maxp, gf, p):
    return pl.pallas_call(
        _head_body,
        out_shape=jax.ShapeDtypeStruct((_B, _NC), jnp.float32),
    )(sums, maxp, gf,
      p['gp_W'], p['gp_b'][None, :], p['gp_ln_g'][None, :], p['gp_ln_b'][None, :],
      p['cm_W1'], p['cm_b1'][None, :], p['cm_ln_g'][None, :], p['cm_ln_b'][None, :],
      p['cm_W2'], p['cm_b2'][None, :], p['th_W'], p['th_b'][None, :])


_MESH = plsc.VectorSubcoreMesh(core_axis_name="c", subcore_axis_name="s")


def _deg_kernel(dst2d, ones16, zeros16):
    @functools.partial(
        pl.kernel, mesh=_MESH,
        compiler_params=pltpu.CompilerParams(use_tc_tiling_on_sc=False,
                                             needs_layout_passes=False),
        out_type=jax.ShapeDtypeStruct((2, _AGG_ROWS, 16), jnp.float32),
        scratch_types=[
            pltpu.VMEM((8, _CH), jnp.int32),
            pltpu.VMEM((_CH, 16), jnp.float32),
            pltpu.VMEM_SHARED((_AGG_ROWS, 16), jnp.float32),
        ],
    )
    def k(dst_hbm, ones_hbm, zz_hbm, out_hbm, idxb, onesb, degsh):
        c = lax.axis_index("c")
        s = lax.axis_index("s")
        pltpu.sync_copy(zz_hbm, degsh.at[pl.ds(s * _ZSL, _ZSL)])
        pltpu.sync_copy(ones_hbm, onesb)
        plsc.subcore_barrier()
        base = c * (_ROWS // 2) + s * _RPT_DEG

        def outer(i, _):
            row0 = base + i * 8
            pltpu.sync_copy(dst_hbm.at[pl.ds(row0, 8)], idxb)
            for j in range(8):
                pltpu.sync_copy(onesb, degsh.at[idxb.at[j]], add=True)
            return 0

        lax.fori_loop(0, _RPT_DEG // 8, outer, 0)
        plsc.subcore_barrier()
        pltpu.sync_copy(degsh.at[pl.ds(s * _ZSL, _ZSL)],
                        out_hbm.at[c, pl.ds(s * _ZSL, _ZSL)])

    return k(dst2d, ones16, zeros16)


def _edge_kernel(l, src2d, dst2d, zcat, ecflat, zeros32):
    @functools.partial(
        pl.kernel, mesh=_MESH,
        compiler_params=pltpu.CompilerParams(use_tc_tiling_on_sc=False,
                                             needs_layout_passes=False),
        out_type=jax.ShapeDtypeStruct((2, _AGG_ROWS, 32), jnp.float32),
        scratch_types=[
            pltpu.VMEM((8, _CH), jnp.int32),
            pltpu.VMEM((8, _CH), jnp.int32),
            pltpu.VMEM((_CH, 32), jnp.float32),
            pltpu.VMEM((_CH, 32), jnp.float32),
            pltpu.VMEM_SHARED((_AGG_ROWS, 32), jnp.float32),
            pltpu.SemaphoreType.DMA,
        ],
    )
    def k(src_hbm, dst_hbm, z_hbm, ec_hbm, zz_hbm, out_hbm,
          srcb, dstb, zrows, ecb, aggsh, sem):
        c = lax.axis_index("c")
        s = lax.axis_index("s")
        pltpu.sync_copy(zz_hbm, aggsh.at[pl.ds(s * _ZSL, _ZSL)])
        plsc.subcore_barrier()
        base = s * _RPT
        ec_base = (2 * l + c) * _EPAD
        zoff = c * _N

        def outer(i, _):
            row0 = base + i * 8
            pltpu.sync_copy(src_hbm.at[pl.ds(row0, 8)], srcb)
            pltpu.sync_copy(dst_hbm.at[pl.ds(row0, 8)], dstb)
            for j in range(8):
                for kk in range(8):
                    srcb[j, pl.ds(kk * 16, 16)] = (
                        srcb[j, pl.ds(kk * 16, 16)] + zoff)
            for j in range(8):
                e0 = (row0 + j) * _CH
                pltpu.async_copy(z_hbm.at[srcb.at[j]], zrows, sem).wait()
                pltpu.sync_copy(ec_hbm.at[pl.ds(ec_base + e0, _CH)], ecb)

                def comp(r, _):
                    for k2 in range(2):
                        sl = pl.ds(k2 * 16, 16)
                        zrows[r, sl] = jnp.maximum(
                            zrows[r, sl] + ecb[r, sl], 0.0)
                    return 0

                lax.fori_loop(0, _CH, comp, 0)
                pltpu.sync_copy(zrows, aggsh.at[dstb.at[j]], add=True)
            return 0

        lax.fori_loop(0, _RPT // 8, outer, 0)
        plsc.subcore_barrier()
        pltpu.sync_copy(aggsh.at[pl.ds(s * _ZSL, _ZSL)],
                        out_hbm.at[c, pl.ds(s * _ZSL, _ZSL)])

    return k(src2d, dst2d, zcat, ecflat, zeros32)


def _maxpool_kernel(hp, batchp):
    @functools.partial(
        pl.kernel, mesh=_MESH,
        compiler_params=pltpu.CompilerParams(use_tc_tiling_on_sc=False,
                                             needs_layout_passes=False),
        out_type=jax.ShapeDtypeStruct((2, _B, _H), jnp.float32),
        scratch_types=[
            pltpu.VMEM((_PHALF, _H), jnp.float32),
            pltpu.VMEM((_PHALF,), jnp.int32),
            pltpu.VMEM((_B, _H), jnp.float32),
            pltpu.VMEM((16, 8, _H), jnp.float32),
            pltpu.VMEM((8, _H), jnp.float32),
            pltpu.VMEM_SHARED((16, _B, _H), jnp.float32),
        ],
    )
    def k(hp_hbm, bp_hbm, out_hbm, hbuf, bbuf, maxb, buf16, outb, shmax):
        c = lax.axis_index("c")
        s = lax.axis_index("s")
        w = c * 16 + s
        iota16 = lax.iota(jnp.int32, 16)
        neg = jnp.full((16,), -1e30, jnp.float32)

        def ini(r, _):
            for cg in range(4):
                maxb[r, pl.ds(cg * 16, 16)] = neg
            return 0

        lax.fori_loop(0, _B, ini, 0)
        base = w * _PPT
        for half in range(2):
            pltpu.sync_copy(hp_hbm.at[pl.ds(base + half * _PHALF, _PHALF)], hbuf)
            pltpu.sync_copy(bp_hbm.at[pl.ds(base + half * _PHALF, _PHALF)], bbuf)

            def body(j, _):
                seg = plsc.load_gather(bbuf, [jnp.zeros((16,), jnp.int32) + j])
                for cg in range(4):
                    col = iota16 + cg * 16
                    cur = plsc.load_gather(maxb, [seg, col])
                    hv = hbuf[j, pl.ds(cg * 16, 16)]
                    plsc.store_scatter(maxb, [seg, col], jnp.maximum(cur, hv))
                return 0

            lax.fori_loop(0, _PHALF, body, 0)
        pltpu.sync_copy(maxb, shmax.at[s])
        plsc.subcore_barrier()

        @pl.when(s < 8)
        def _():
            for t2 in range(16):
                pltpu.sync_copy(shmax.at[t2, pl.ds(8 * s, 8)], buf16.at[t2])
            for si in range(8):
                for cg in range(4):
                    sl = pl.ds(cg * 16, 16)
                    acc = buf16[0, si, sl]
                    for t2 in range(1, 16):
                        acc = jnp.maximum(acc, buf16[t2, si, sl])
                    outb[si, sl] = acc
            pltpu.sync_copy(outb, out_hbm.at[c, pl.ds(8 * s, 8)])

    return k(hp, batchp)


def kernel(x, edge_index, edge_attr, edge_gate_type, batch, global_features,
           params):
    p = params
    src = edge_index[0]
    dst = edge_index[1]
    pad = _EPAD - _E
    src2d = jnp.concatenate([src, jnp.zeros((pad,), jnp.int32)]).reshape(_ROWS, _CH)
    dst2d = jnp.concatenate([dst, jnp.full((pad,), _N, jnp.int32)]).reshape(_ROWS, _CH)
    ea_p = jnp.concatenate([edge_attr, jnp.zeros((pad, _EF), jnp.float32)])
    gt_p = jnp.concatenate([edge_gate_type.astype(jnp.float32),
                            jnp.zeros((pad,), jnp.float32)])[:, None]

    we_s = jnp.concatenate([p['mp%d_Wm' % l][_H:_H + _EF] for l in range(_L)],
                           axis=1)
    gemb_s = jnp.stack([p['mp%d_gate_emb' % l] for l in range(_L)])
    wg_s = jnp.stack([p['mp%d_Wm' % l][_H + _EF:] for l in range(_L)])
    bm_s = jnp.concatenate([p['mp%d_bm' % l] for l in range(_L)])[None, :]

    zeros32 = jnp.zeros((_ZSL, 32), jnp.float32)
    zeros16 = jnp.zeros((_ZSL, 16), jnp.float32)
    ones16 = jnp.ones((_CH, 16), jnp.float32)

    h = _embed(x, p['ne_W'], p['ne_b'][None, :], p['ne_ln_g'][None, :],
               p['ne_ln_b'][None, :])
    ec = _ec_all(ea_p, gt_p, we_s, gemb_s, wg_s, bm_s)
    ecflat = ec.reshape(2 * _L * _EPAD, 32)
    degp = _deg_kernel(dst2d, ones16, zeros16)

    for l in range(_L):
        zcat = _z_halves(h, p['mp%d_Wm' % l][:_H]).reshape(2 * _N, 32)
        agg = _edge_kernel(l, src2d, dst2d, zcat, ecflat, zeros32)
        h = _update(h, agg, degp, p['mp%d_Wu' % l], p['mp%d_bu' % l][None, :])

    batchf = batch.astype(jnp.float32)[:, None]
    sums = _pool_sum(h, batchf)
    hp = jnp.concatenate([h, jnp.full((_NPOOL - _N, _H), -1e30, jnp.float32)])
    batchp = jnp.concatenate([batch, jnp.full((_NPOOL - _N,), _B - 1, jnp.int32)])
    maxp = _maxpool_kernel(hp, batchp)
    return _head(sums, maxp, global_features, p)


# speedup vs baseline: 3.6120x; 1.6621x over previous
import functools

import jax
import jax.numpy as jnp
from jax import lax
from jax.experimental import pallas as pl
from jax.experimental.pallas import tpu as pltpu
from jax.experimental.pallas import tpu_sc as plsc

_N = 50000
_E = 800000
_B = 64
_NF = 128
_EF = 4
_GF = 52
_H = 64
_L = 4
_NC = 9
_NG = 8
_GE = 16

_CH = 128
_EPAD = 819200
_ROWS = _EPAD // _CH
_RPT = _ROWS // 16
_RPT_DEG = _ROWS // 32
_AGG_ROWS = 50048
_ZSL = _AGG_ROWS // 16
_NPOOL = 50176
_PPT = _NPOOL // 32
_PHALF = _PPT // 2


def _embed_z_body(x_ref, w_ref, b_ref, g_ref, bb_ref, wz_ref, o_ref, oz_ref):
    h = jnp.maximum(jnp.dot(x_ref[...], w_ref[...],
                            preferred_element_type=jnp.float32) + b_ref[...], 0.0)
    mu = jnp.mean(h, axis=-1, keepdims=True)
    var = jnp.mean((h - mu) ** 2, axis=-1, keepdims=True)
    h = (h - mu) * lax.rsqrt(var + 1e-5) * g_ref[...] + bb_ref[...]
    o_ref[...] = h
    z = jnp.dot(h, wz_ref[...], preferred_element_type=jnp.float32)
    oz_ref[0] = z[:, :32]
    oz_ref[1] = z[:, 32:]


def _embed_z(x, w, b, g, bb, wz):
    grid = _N // 1000
    return pl.pallas_call(
        _embed_z_body,
        grid=(grid,),
        in_specs=[
            pl.BlockSpec((1000, _NF), lambda i: (i, 0)),
            pl.BlockSpec((_NF, _H), lambda i: (0, 0)),
            pl.BlockSpec((1, _H), lambda i: (0, 0)),
            pl.BlockSpec((1, _H), lambda i: (0, 0)),
            pl.BlockSpec((1, _H), lambda i: (0, 0)),
            pl.BlockSpec((_H, _H), lambda i: (0, 0)),
        ],
        out_specs=[pl.BlockSpec((1000, _H), lambda i: (i, 0)),
                   pl.BlockSpec((2, 1000, 32), lambda i: (0, i, 0))],
        out_shape=[jax.ShapeDtypeStruct((_N, _H), jnp.float32),
                   jax.ShapeDtypeStruct((2, _N, 32), jnp.float32)],
    )(x, w, b, g, bb, wz)


def _gproj_body(ge_ref, wg_ref, bm_ref, o_ref):
    for l in range(_L):
        o_ref[l] = (jnp.dot(ge_ref[l], wg_ref[l],
                            preferred_element_type=jnp.float32) + bm_ref[l])


def _gproj(gemb_s, wg_s, bm4):
    return pl.pallas_call(
        _gproj_body,
        out_shape=jax.ShapeDtypeStruct((_L, _NG, _H), jnp.float32),
    )(gemb_s, wg_s, bm4)


def _upd_core(h_ref, agg_ref, deg_ref, w1_ref, w2a_ref, w2b_ref, b_ref):
    deg = deg_ref[0, :, :1] + deg_ref[1, :, :1]
    inv = 1.0 / jnp.maximum(deg, 1.0)
    a0 = agg_ref[0] * inv
    a1 = agg_ref[1] * inv
    u = (jnp.dot(h_ref[...], w1_ref[...], preferred_element_type=jnp.float32)
         + jnp.dot(a0, w2a_ref[...], preferred_element_type=jnp.float32)
         + jnp.dot(a1, w2b_ref[...], preferred_element_type=jnp.float32)
         + b_ref[...])
    return h_ref[...] + jnp.maximum(u, 0.0)


def _upd_z_body(h_ref, agg_ref, deg_ref, w1_ref, w2a_ref, w2b_ref, b_ref,
                wz_ref, o_ref, oz_ref):
    h = _upd_core(h_ref, agg_ref, deg_ref, w1_ref, w2a_ref, w2b_ref, b_ref)
    o_ref[...] = h
    z = jnp.dot(h, wz_ref[...], preferred_element_type=jnp.float32)
    oz_ref[0] = z[:, :32]
    oz_ref[1] = z[:, 32:]


def _upd_body(h_ref, agg_ref, deg_ref, w1_ref, w2a_ref, w2b_ref, b_ref, o_ref):
    o_ref[...] = _upd_core(h_ref, agg_ref, deg_ref, w1_ref, w2a_ref, w2b_ref,
                           b_ref)


def _update(h, agg, degp, wu, bu, wz=None):
    grid = _N // 1000
    in_specs = [
        pl.BlockSpec((1000, _H), lambda i: (i, 0)),
        pl.BlockSpec((2, 1000, 32), lambda i: (0, i, 0)),
        pl.BlockSpec((2, 1000, 16), lambda i: (0, i, 0)),
        pl.BlockSpec((_H, _H), lambda i: (0, 0)),
        pl.BlockSpec((32, _H), lambda i: (0, 0)),
        pl.BlockSpec((32, _H), lambda i: (0, 0)),
        pl.BlockSpec((1, _H), lambda i: (0, 0)),
    ]
    args = [h, agg, degp, wu[:_H], wu[_H:_H + 32], wu[_H + 32:], bu]
    if wz is None:
        return pl.pallas_call(
            _upd_body,
            grid=(grid,),
            in_specs=in_specs,
            out_specs=pl.BlockSpec((1000, _H), lambda i: (i, 0)),
            out_shape=jax.ShapeDtypeStruct((_N, _H), jnp.float32),
        )(*args)
    return pl.pallas_call(
        _upd_z_body,
        grid=(grid,),
        in_specs=in_specs + [pl.BlockSpec((_H, _H), lambda i: (0, 0))],
        out_specs=[pl.BlockSpec((1000, _H), lambda i: (i, 0)),
                   pl.BlockSpec((2, 1000, 32), lambda i: (0, i, 0))],
        out_shape=[jax.ShapeDtypeStruct((_N, _H), jnp.float32),
                   jax.ShapeDtypeStruct((2, _N, 32), jnp.float32)],
    )(*args, wz)


def _psum_body(h_ref, b_ref, o_ref):
    i = pl.program_id(0)

    @pl.when(i == 0)
    def _():
        o_ref[...] = jnp.zeros_like(o_ref)

    p = (b_ref[...] == jnp.arange(_B, dtype=jnp.int32)[None, :]
         .astype(jnp.float32)).astype(jnp.float32)
    h1 = jnp.concatenate(
        [h_ref[...], jnp.ones((1000, 1), jnp.float32)], axis=1)
    o_ref[...] += lax.dot_general(p, h1, (((0,), (0,)), ((), ())),
                                  preferred_element_type=jnp.float32)


def _pool_sum(h, batchf):
    grid = _N // 1000
    return pl.pallas_call(
        _psum_body,
        grid=(grid,),
        in_specs=[
            pl.BlockSpec((1000, _H), lambda i: (i, 0)),
            pl.BlockSpec((1000, 1), lambda i: (i, 0)),
        ],
        out_specs=pl.BlockSpec((_B, _H + 1), lambda i: (0, 0)),
        out_shape=jax.ShapeDtypeStruct((_B, _H + 1), jnp.float32),
    )(h, batchf)


def _ln(v, g, b):
    mu = jnp.mean(v, axis=-1, keepdims=True)
    var = jnp.mean((v - mu) ** 2, axis=-1, keepdims=True)
    return (v - mu) * lax.rsqrt(var + 1e-5) * g + b


def _head_body(sums_ref, maxp_ref, gf_ref, gpw_ref, gpb_ref, gpg_ref, gpbb_ref,
               w1_ref, b1_ref, lng_ref, lnb_ref, w2_ref, b2_ref, wt_ref,
               bt_ref, o_ref):
    counts = sums_ref[:, _H:_H + 1]
    h_sum = sums_ref[:, :_H]
    h_mean = h_sum / jnp.maximum(counts, 1.0)
    h_max = jnp.maximum(maxp_ref[0], maxp_ref[1])
    h_max = jnp.where(counts > 0.0, h_max, 0.0)
    g = jnp.maximum(jnp.dot(gf_ref[...], gpw_ref[...],
                            preferred_element_type=jnp.float32) + gpb_ref[...], 0.0)
    g = _ln(g, gpg_ref[...], gpbb_ref[...])
    c = jnp.concatenate([h_mean, h_max, h_sum, g], axis=-1)
    c = jnp.maximum(jnp.dot(c, w1_ref[...],
                            preferred_element_type=jnp.float32) + b1_ref[...], 0.0)
    c = _ln(c, lng_ref[...], lnb_ref[...])
    c = jnp.maximum(jnp.dot(c, w2_ref[...],
                            preferred_element_type=jnp.float32) + b2_ref[...], 0.0)
    o_ref[...] = jnp.dot(c, wt_ref[...],
                         preferred_element_type=jnp.float32) + bt_ref[...]


def _head(sums, maxp, gf, p):
    return pl.pallas_call(
        _head_body,
        out_shape=jax.ShapeDtypeStruct((_B, _NC), jnp.float32),
    )(sums, maxp, gf,
      p['gp_W'], p['gp_b'][None, :], p['gp_ln_g'][None, :], p['gp_ln_b'][None, :],
      p['cm_W1'], p['cm_b1'][None, :], p['cm_ln_g'][None, :], p['cm_ln_b'][None, :],
      p['cm_W2'], p['cm_b2'][None, :], p['th_W'], p['th_b'][None, :])


_MESH = plsc.VectorSubcoreMesh(core_axis_name="c", subcore_axis_name="s")


def _deg_kernel(dst2d, ones16, zeros16):
    @functools.partial(
        pl.kernel, mesh=_MESH,
        compiler_params=pltpu.CompilerParams(use_tc_tiling_on_sc=False,
                                             needs_layout_passes=False),
        out_type=jax.ShapeDtypeStruct((2, _AGG_ROWS, 16), jnp.float32),
        scratch_types=[
            pltpu.VMEM((8, _CH), jnp.int32),
            pltpu.VMEM((_CH, 16), jnp.float32),
            pltpu.VMEM_SHARED((_AGG_ROWS, 16), jnp.float32),
        ],
    )
    def k(dst_hbm, ones_hbm, zz_hbm, out_hbm, idxb, onesb, degsh):
        c = lax.axis_index("c")
        s = lax.axis_index("s")
        pltpu.sync_copy(zz_hbm, degsh.at[pl.ds(s * _ZSL, _ZSL)])
        pltpu.sync_copy(ones_hbm, onesb)
        plsc.subcore_barrier()
        base = c * (_ROWS // 2) + s * _RPT_DEG

        def outer(i, _):
            row0 = base + i * 8
            pltpu.sync_copy(dst_hbm.at[pl.ds(row0, 8)], idxb)
            for j in range(8):
                pltpu.sync_copy(onesb, degsh.at[idxb.at[j]], add=True)
            return 0

        lax.fori_loop(0, _RPT_DEG // 8, outer, 0)
        plsc.subcore_barrier()
        pltpu.sync_copy(degsh.at[pl.ds(s * _ZSL, _ZSL)],
                        out_hbm.at[c, pl.ds(s * _ZSL, _ZSL)])

    return k(dst2d, ones16, zeros16)


_NBR = 16
_NBATCH = _RPT // _NBR


def _edge_kernel(src2d, dst2d, gt2d, zcat, ea2, wh, gp, zeros32):
    @functools.partial(
        pl.kernel, mesh=_MESH,
        compiler_params=pltpu.CompilerParams(use_tc_tiling_on_sc=False,
                                             needs_layout_passes=False),
        out_type=jax.ShapeDtypeStruct((2, _AGG_ROWS, 32), jnp.float32),
        scratch_types=[
            pltpu.VMEM((_NBR, _CH), jnp.int32),
            pltpu.VMEM((_NBR, _CH), jnp.int32),
            pltpu.VMEM((_NBR, _CH), jnp.int32),
            pltpu.VMEM((_CH, 32), jnp.float32),
            pltpu.VMEM((_CH, 32), jnp.float32),
            pltpu.VMEM((_CH, 32), jnp.float32),
            pltpu.VMEM((_CH, 32), jnp.float32),
            pltpu.VMEM((_CH, _EF), jnp.float32),
            pltpu.VMEM((_CH, _EF), jnp.float32),
            pltpu.VMEM((_CH, _EF), jnp.float32),
            pltpu.VMEM((_CH, _EF), jnp.float32),
            pltpu.VMEM((_EF, 32), jnp.float32),
            pltpu.VMEM((_NG, 32), jnp.float32),
            pltpu.VMEM_SHARED((_AGG_ROWS, 32), jnp.float32),
            pltpu.SemaphoreType.DMA,
            pltpu.SemaphoreType.DMA,
            pltpu.SemaphoreType.DMA,
            pltpu.SemaphoreType.DMA,
            pltpu.SemaphoreType.DMA,
            pltpu.SemaphoreType.DMA,
            pltpu.SemaphoreType.DMA,
            pltpu.SemaphoreType.DMA,
        ],
    )
    def k(src_hbm, dst_hbm, gt_hbm, z_hbm, ea_hbm, wh_hbm, gp_hbm, zz_hbm,
          out_hbm, srcb, dstb, gtb, zr0, zr1, zr2, zr3, ea0b, ea1b, ea2b,
          ea3b, wv, gv, aggsh, gs0, gs1, gs2, gs3, ss0, ss1, ss2, ss3):
        c = lax.axis_index("c")
        s = lax.axis_index("s")
        pltpu.sync_copy(zz_hbm, aggsh.at[pl.ds(s * _ZSL, _ZSL)])
        pltpu.sync_copy(wh_hbm.at[c], wv)
        pltpu.sync_copy(gp_hbm.at[c], gv)
        plsc.subcore_barrier()
        iota16 = lax.iota(jnp.int32, 16)
        zero16 = jnp.zeros((16,), jnp.int32)
        wvec = [(wv[kk, pl.ds(0, 16)], wv[kk, pl.ds(16, 16)])
                for kk in range(_EF)]
        zrs = [zr0, zr1, zr2, zr3]
        eas = [ea0b, ea1b, ea2b, ea3b]
        gsems = [gs0, gs1, gs2, gs3]
        ssems = [ss0, ss1, ss2, ss3]
        zoff = c * _N
        base = s * _RPT

        def drain(sem, buf):
            pltpu.make_async_copy(zz_hbm.at[pl.ds(0, _CH)], buf, sem).wait()

        def drain_ea(sem, buf):
            pltpu.make_async_copy(ea_hbm.at[pl.ds(0, _CH)], buf, sem).wait()

        def fire(slot, jj, row0):
            pltpu.async_copy(z_hbm.at[srcb.at[jj]], zrs[slot], gsems[slot])
            pltpu.async_copy(ea_hbm.at[pl.ds((row0 + jj) * _CH, _CH)],
                             eas[slot], gsems[slot])

        def compute(jj, zr, eab):
            def comp(e, _):
                jv = zero16 + jj
                ev = zero16 + e
                gt_b = plsc.load_gather(gtb, [jv, ev])
                ea0 = plsc.load_gather(eab, [ev, zero16])
                ea1 = plsc.load_gather(eab, [ev, zero16 + 1])
                ea2 = plsc.load_gather(eab, [ev, zero16 + 2])
                ea3 = plsc.load_gather(eab, [ev, zero16 + 3])
                g0 = plsc.load_gather(gv, [gt_b, iota16])
                g1 = plsc.load_gather(gv, [gt_b, iota16 + 16])
                acc0 = (g0 + wvec[0][0] * ea0 + wvec[1][0] * ea1
                        + wvec[2][0] * ea2 + wvec[3][0] * ea3)
                acc1 = (g1 + wvec[0][1] * ea0 + wvec[1][1] * ea1
                        + wvec[2][1] * ea2 + wvec[3][1] * ea3)
                s0 = pl.ds(0, 16)
                s1 = pl.ds(16, 16)
                zr[e, s0] = jnp.maximum(acc0 + zr[e, s0], 0.0)
                zr[e, s1] = jnp.maximum(acc1 + zr[e, s1], 0.0)
                return 0

            lax.fori_loop(0, _CH, comp, 0)

        def batch(b, _):
            row0 = base + b * _NBR
            pltpu.sync_copy(src_hbm.at[pl.ds(row0, _NBR)], srcb)
            pltpu.sync_copy(dst_hbm.at[pl.ds(row0, _NBR)], dstb)
            pltpu.sync_copy(gt_hbm.at[pl.ds(row0, _NBR)], gtb)

            def adj(r, _):
                for kk in range(_CH // 16):
                    sl = pl.ds(kk * 16, 16)
                    srcb[r, sl] = srcb[r, sl] + zoff
                return 0

            lax.fori_loop(0, _NBR, adj, 0)

            @pl.when(b > 0)
            def _():
                for r in range(4):
                    drain(ssems[r], zrs[r])

            for r in range(3):
                fire(r, r, row0)

            def grp(g, _):
                for r in range(4):
                    jj = g * 4 + r
                    drain(gsems[r], zrs[r])
                    drain_ea(gsems[r], eas[r])
                    compute(jj, zrs[r], eas[r])
                    pltpu.async_copy(zrs[r], aggsh.at[dstb.at[jj]],
                                     ssems[r], add=True)
                    nslot = (r + 3) % 4

                    @pl.when(jnp.logical_and(jj >= 1, jj < _NBR - 3))
                    def _():
                        drain(ssems[nslot], zrs[nslot])

                    @pl.when(jj < _NBR - 3)
                    def _():
                        fire(nslot, jj + 3, row0)

                return 0

            lax.fori_loop(0, _NBR // 4, grp, 0)
            return 0

        lax.fori_loop(0, _NBATCH, batch, 0)
        for r in range(4):
            drain(ssems[r], zrs[r])
        plsc.subcore_barrier()
        pltpu.sync_copy(aggsh.at[pl.ds(s * _ZSL, _ZSL)],
                        out_hbm.at[c, pl.ds(s * _ZSL, _ZSL)])

    return k(src2d, dst2d, gt2d, zcat, ea2, wh, gp, zeros32)


def _maxpool_kernel(hp, batchp):
    @functools.partial(
        pl.kernel, mesh=_MESH,
        compiler_params=pltpu.CompilerParams(use_tc_tiling_on_sc=False,
                                             needs_layout_passes=False),
        out_type=jax.ShapeDtypeStruct((2, _B, _H), jnp.float32),
        scratch_types=[
            pltpu.VMEM((_PHALF, _H), jnp.float32),
            pltpu.VMEM((_PHALF,), jnp.int32),
            pltpu.VMEM((_B, _H), jnp.float32),
            pltpu.VMEM((16, 8, _H), jnp.float32),
            pltpu.VMEM((8, _H), jnp.float32),
            pltpu.VMEM_SHARED((16, _B, _H), jnp.float32),
        ],
    )
    def k(hp_hbm, bp_hbm, out_hbm, hbuf, bbuf, maxb, buf16, outb, shmax):
        c = lax.axis_index("c")
        s = lax.axis_index("s")
        w = c * 16 + s
        iota16 = lax.iota(jnp.int32, 16)
        neg = jnp.full((16,), -1e30, jnp.float32)

        def ini(r, _):
            for cg in range(4):
                maxb[r, pl.ds(cg * 16, 16)] = neg
            return 0

        lax.fori_loop(0, _B, ini, 0)
        base = w * _PPT
        for half in range(2):
            pltpu.sync_copy(hp_hbm.at[pl.ds(base + half * _PHALF, _PHALF)], hbuf)
            pltpu.sync_copy(bp_hbm.at[pl.ds(base + half * _PHALF, _PHALF)], bbuf)

            def body(j, _):
                seg = plsc.load_gather(bbuf, [jnp.zeros((16,), jnp.int32) + j])
                for cg in range(4):
                    col = iota16 + cg * 16
                    cur = plsc.load_gather(maxb, [seg, col])
                    hv = hbuf[j, pl.ds(cg * 16, 16)]
                    plsc.store_scatter(maxb, [seg, col], jnp.maximum(cur, hv))
                return 0

            lax.fori_loop(0, _PHALF, body, 0)
        pltpu.sync_copy(maxb, shmax.at[s])
        plsc.subcore_barrier()

        @pl.when(s < 8)
        def _():
            for t2 in range(16):
                pltpu.sync_copy(shmax.at[t2, pl.ds(8 * s, 8)], buf16.at[t2])
            for si in range(8):
                for cg in range(4):
                    sl = pl.ds(cg * 16, 16)
                    acc = buf16[0, si, sl]
                    for t2 in range(1, 16):
                        acc = jnp.maximum(acc, buf16[t2, si, sl])
                    outb[si, sl] = acc
            pltpu.sync_copy(outb, out_hbm.at[c, pl.ds(8 * s, 8)])

    return k(hp, batchp)


def kernel(x, edge_index, edge_attr, edge_gate_type, batch, global_features,
           params):
    p = params
    src = edge_index[0]
    dst = edge_index[1]
    pad = _EPAD - _E
    src2d = jnp.concatenate([src, jnp.zeros((pad,), jnp.int32)]).reshape(_ROWS, _CH)
    dst2d = jnp.concatenate([dst, jnp.full((pad,), _N, jnp.int32)]).reshape(_ROWS, _CH)
    gt2d = jnp.concatenate([edge_gate_type,
                            jnp.zeros((pad,), jnp.int32)]).reshape(_ROWS, _CH)
    ea_p = jnp.concatenate([edge_attr, jnp.zeros((pad, _EF), jnp.float32)])

    gemb_s = jnp.stack([p['mp%d_gate_emb' % l] for l in range(_L)])
    wg_s = jnp.stack([p['mp%d_Wm' % l][_H + _EF:] for l in range(_L)])
    bm4 = jnp.stack([p['mp%d_bm' % l] for l in range(_L)])[:, None, :]
    bm4 = jnp.broadcast_to(bm4, (_L, _NG, _H))
    gp_all = _gproj(gemb_s, wg_s, bm4)
    whs = []
    gps = []
    for l in range(_L):
        we = p['mp%d_Wm' % l][_H:_H + _EF]
        whs.append(jnp.stack([we[:, :32], we[:, 32:]]))
        gps.append(jnp.stack([gp_all[l][:, :32], gp_all[l][:, 32:]]))

    zeros32 = jnp.zeros((_ZSL, 32), jnp.float32)
    zeros16 = jnp.zeros((_ZSL, 16), jnp.float32)
    ones16 = jnp.ones((_CH, 16), jnp.float32)

    h, zhalves = _embed_z(x, p['ne_W'], p['ne_b'][None, :], p['ne_ln_g'][None, :],
                          p['ne_ln_b'][None, :], p['mp0_Wm'][:_H])
    degp = _deg_kernel(dst2d, ones16, zeros16)

    for l in range(_L):
        zcat = zhalves.reshape(2 * _N, 32)
        agg = _edge_kernel(src2d, dst2d, gt2d, zcat, ea_p, whs[l], gps[l],
                           zeros32)
        if l < _L - 1:
            h, zhalves = _update(h, agg, degp, p['mp%d_Wu' % l],
                                 p['mp%d_bu' % l][None, :],
                                 wz=p['mp%d_Wm' % (l + 1)][:_H])
        else:
            h = _update(h, agg, degp, p['mp%d_Wu' % l],
                        p['mp%d_bu' % l][None, :])

    batchf = batch.astype(jnp.float32)[:, None]
    sums = _pool_sum(h, batchf)
    hp = jnp.concatenate([h, jnp.full((_NPOOL - _N, _H), -1e30, jnp.float32)])
    batchp = jnp.concatenate([batch, jnp.full((_NPOOL - _N,), _B - 1, jnp.int32)])
    maxp = _maxpool_kernel(hp, batchp)
    return _head(sums, maxp, global_features, p)


# speedup vs baseline: 7.2325x; 2.0023x over previous
import functools

import jax
import jax.numpy as jnp
from jax import lax
from jax.experimental import pallas as pl
from jax.experimental.pallas import tpu as pltpu
from jax.experimental.pallas import tpu_sc as plsc

_N = 50000
_E = 800000
_B = 64
_NF = 128
_EF = 4
_GF = 52
_H = 64
_L = 4
_NC = 9
_NG = 8
_GE = 16

_CH = 128
_EPAD = 819200
_ROWS = _EPAD // _CH
_RPT = _ROWS // 16
_RPT_DEG = _ROWS // 32
_AGG_ROWS = 50048
_ZSL = _AGG_ROWS // 16
_NPOOL = 50176
_PPT = _NPOOL // 32
_PHALF = _PPT // 2


def _embed_z_body(x_ref, w_ref, b_ref, g_ref, bb_ref, wz_ref, o_ref, oz_ref):
    h = jnp.maximum(jnp.dot(x_ref[...], w_ref[...],
                            preferred_element_type=jnp.float32) + b_ref[...], 0.0)
    mu = jnp.mean(h, axis=-1, keepdims=True)
    var = jnp.mean((h - mu) ** 2, axis=-1, keepdims=True)
    h = (h - mu) * lax.rsqrt(var + 1e-5) * g_ref[...] + bb_ref[...]
    o_ref[...] = h
    z = jnp.dot(h, wz_ref[...], preferred_element_type=jnp.float32)
    oz_ref[0] = z[:, :32]
    oz_ref[1] = z[:, 32:]


def _embed_z(x, w, b, g, bb, wz):
    grid = _N // 1000
    return pl.pallas_call(
        _embed_z_body,
        grid=(grid,),
        in_specs=[
            pl.BlockSpec((1000, _NF), lambda i: (i, 0)),
            pl.BlockSpec((_NF, _H), lambda i: (0, 0)),
            pl.BlockSpec((1, _H), lambda i: (0, 0)),
            pl.BlockSpec((1, _H), lambda i: (0, 0)),
            pl.BlockSpec((1, _H), lambda i: (0, 0)),
            pl.BlockSpec((_H, _H), lambda i: (0, 0)),
        ],
        out_specs=[pl.BlockSpec((1000, _H), lambda i: (i, 0)),
                   pl.BlockSpec((2, 1000, 32), lambda i: (0, i, 0))],
        out_shape=[jax.ShapeDtypeStruct((_N, _H), jnp.float32),
                   jax.ShapeDtypeStruct((2, _N, 32), jnp.float32)],
    )(x, w, b, g, bb, wz)


def _gproj_body(ge_ref, wg_ref, bm_ref, o_ref):
    for l in range(_L):
        o_ref[l] = (jnp.dot(ge_ref[l], wg_ref[l],
                            preferred_element_type=jnp.float32) + bm_ref[l])


def _gproj(gemb_s, wg_s, bm4):
    return pl.pallas_call(
        _gproj_body,
        out_shape=jax.ShapeDtypeStruct((_L, _NG, _H), jnp.float32),
    )(gemb_s, wg_s, bm4)


def _upd_core(h_ref, agg_ref, deg_ref, w1_ref, w2a_ref, w2b_ref, b_ref):
    deg = deg_ref[0, :, :1] + deg_ref[1, :, :1]
    inv = 1.0 / jnp.maximum(deg, 1.0)
    a0 = agg_ref[0] * inv
    a1 = agg_ref[1] * inv
    u = (jnp.dot(h_ref[...], w1_ref[...], preferred_element_type=jnp.float32)
         + jnp.dot(a0, w2a_ref[...], preferred_element_type=jnp.float32)
         + jnp.dot(a1, w2b_ref[...], preferred_element_type=jnp.float32)
         + b_ref[...])
    return h_ref[...] + jnp.maximum(u, 0.0)


def _upd_z_body(h_ref, agg_ref, deg_ref, w1_ref, w2a_ref, w2b_ref, b_ref,
                wz_ref, o_ref, oz_ref):
    h = _upd_core(h_ref, agg_ref, deg_ref, w1_ref, w2a_ref, w2b_ref, b_ref)
    o_ref[...] = h
    z = jnp.dot(h, wz_ref[...], preferred_element_type=jnp.float32)
    oz_ref[0] = z[:, :32]
    oz_ref[1] = z[:, 32:]


def _upd_body(h_ref, agg_ref, deg_ref, w1_ref, w2a_ref, w2b_ref, b_ref, o_ref):
    o_ref[...] = _upd_core(h_ref, agg_ref, deg_ref, w1_ref, w2a_ref, w2b_ref,
                           b_ref)


def _update(h, agg, degp, wu, bu, wz=None):
    grid = _N // 1000
    in_specs = [
        pl.BlockSpec((1000, _H), lambda i: (i, 0)),
        pl.BlockSpec((2, 1000, 32), lambda i: (0, i, 0)),
        pl.BlockSpec((2, 1000, 16), lambda i: (0, i, 0)),
        pl.BlockSpec((_H, _H), lambda i: (0, 0)),
        pl.BlockSpec((32, _H), lambda i: (0, 0)),
        pl.BlockSpec((32, _H), lambda i: (0, 0)),
        pl.BlockSpec((1, _H), lambda i: (0, 0)),
    ]
    args = [h, agg, degp, wu[:_H], wu[_H:_H + 32], wu[_H + 32:], bu]
    if wz is None:
        return pl.pallas_call(
            _upd_body,
            grid=(grid,),
            in_specs=in_specs,
            out_specs=pl.BlockSpec((1000, _H), lambda i: (i, 0)),
            out_shape=jax.ShapeDtypeStruct((_N, _H), jnp.float32),
        )(*args)
    return pl.pallas_call(
        _upd_z_body,
        grid=(grid,),
        in_specs=in_specs + [pl.BlockSpec((_H, _H), lambda i: (0, 0))],
        out_specs=[pl.BlockSpec((1000, _H), lambda i: (i, 0)),
                   pl.BlockSpec((2, 1000, 32), lambda i: (0, i, 0))],
        out_shape=[jax.ShapeDtypeStruct((_N, _H), jnp.float32),
                   jax.ShapeDtypeStruct((2, _N, 32), jnp.float32)],
    )(*args, wz)


def _psum_body(h_ref, b_ref, o_ref):
    i = pl.program_id(0)

    @pl.when(i == 0)
    def _():
        o_ref[...] = jnp.zeros_like(o_ref)

    p = (b_ref[...] == jnp.arange(_B, dtype=jnp.int32)[None, :]
         .astype(jnp.float32)).astype(jnp.float32)
    h1 = jnp.concatenate(
        [h_ref[...], jnp.ones((1000, 1), jnp.float32)], axis=1)
    o_ref[...] += lax.dot_general(p, h1, (((0,), (0,)), ((), ())),
                                  preferred_element_type=jnp.float32)


def _pool_sum(h, batchf):
    grid = _N // 1000
    return pl.pallas_call(
        _psum_body,
        grid=(grid,),
        in_specs=[
            pl.BlockSpec((1000, _H), lambda i: (i, 0)),
            pl.BlockSpec((1000, 1), lambda i: (i, 0)),
        ],
        out_specs=pl.BlockSpec((_B, _H + 1), lambda i: (0, 0)),
        out_shape=jax.ShapeDtypeStruct((_B, _H + 1), jnp.float32),
    )(h, batchf)


def _ln(v, g, b):
    mu = jnp.mean(v, axis=-1, keepdims=True)
    var = jnp.mean((v - mu) ** 2, axis=-1, keepdims=True)
    return (v - mu) * lax.rsqrt(var + 1e-5) * g + b


def _head_body(sums_ref, maxp_ref, gf_ref, gpw_ref, gpb_ref, gpg_ref, gpbb_ref,
               w1_ref, b1_ref, lng_ref, lnb_ref, w2_ref, b2_ref, wt_ref,
               bt_ref, o_ref):
    counts = sums_ref[:, _H:_H + 1]
    h_sum = sums_ref[:, :_H]
    h_mean = h_sum / jnp.maximum(counts, 1.0)
    h_max = jnp.maximum(maxp_ref[0], maxp_ref[1])
    h_max = jnp.where(counts > 0.0, h_max, 0.0)
    g = jnp.maximum(jnp.dot(gf_ref[...], gpw_ref[...],
                            preferred_element_type=jnp.float32) + gpb_ref[...], 0.0)
    g = _ln(g, gpg_ref[...], gpbb_ref[...])
    c = jnp.concatenate([h_mean, h_max, h_sum, g], axis=-1)
    c = jnp.maximum(jnp.dot(c, w1_ref[...],
                            preferred_element_type=jnp.float32) + b1_ref[...], 0.0)
    c = _ln(c, lng_ref[...], lnb_ref[...])
    c = jnp.maximum(jnp.dot(c, w2_ref[...],
                            preferred_element_type=jnp.float32) + b2_ref[...], 0.0)
    o_ref[...] = jnp.dot(c, wt_ref[...],
                         preferred_element_type=jnp.float32) + bt_ref[...]


def _head(sums, maxp, gf, p):
    return pl.pallas_call(
        _head_body,
        out_shape=jax.ShapeDtypeStruct((_B, _NC), jnp.float32),
    )(sums, maxp, gf,
      p['gp_W'], p['gp_b'][None, :], p['gp_ln_g'][None, :], p['gp_ln_b'][None, :],
      p['cm_W1'], p['cm_b1'][None, :], p['cm_ln_g'][None, :], p['cm_ln_b'][None, :],
      p['cm_W2'], p['cm_b2'][None, :], p['th_W'], p['th_b'][None, :])


_MESH = plsc.VectorSubcoreMesh(core_axis_name="c", subcore_axis_name="s")


def _deg_kernel(dst2d, ones16, zeros16):
    @functools.partial(
        pl.kernel, mesh=_MESH,
        compiler_params=pltpu.CompilerParams(use_tc_tiling_on_sc=False,
                                             needs_layout_passes=False),
        out_type=jax.ShapeDtypeStruct((2, _AGG_ROWS, 16), jnp.float32),
        scratch_types=[
            pltpu.VMEM((8, _CH), jnp.int32),
            pltpu.VMEM((_CH, 16), jnp.float32),
            pltpu.VMEM_SHARED((_AGG_ROWS, 16), jnp.float32),
        ],
    )
    def k(dst_hbm, ones_hbm, zz_hbm, out_hbm, idxb, onesb, degsh):
        c = lax.axis_index("c")
        s = lax.axis_index("s")
        pltpu.sync_copy(zz_hbm, degsh.at[pl.ds(s * _ZSL, _ZSL)])
        pltpu.sync_copy(ones_hbm, onesb)
        plsc.subcore_barrier()
        base = c * (_ROWS // 2) + s * _RPT_DEG

        def outer(i, _):
            row0 = base + i * 8
            pltpu.sync_copy(dst_hbm.at[pl.ds(row0, 8)], idxb)
            for j in range(8):
                pltpu.sync_copy(onesb, degsh.at[idxb.at[j]], add=True)
            return 0

        lax.fori_loop(0, _RPT_DEG // 8, outer, 0)
        plsc.subcore_barrier()
        pltpu.sync_copy(degsh.at[pl.ds(s * _ZSL, _ZSL)],
                        out_hbm.at[c, pl.ds(s * _ZSL, _ZSL)])

    return k(dst2d, ones16, zeros16)


_NBR = 16
_NBATCH = _RPT // _NBR


def _edge_kernel(src2d, dst2d, gt2d, zcat, ea2, wh, gp, zeros32):
    @functools.partial(
        pl.kernel, mesh=_MESH,
        compiler_params=pltpu.CompilerParams(use_tc_tiling_on_sc=False,
                                             needs_layout_passes=False),
        out_type=jax.ShapeDtypeStruct((2, _AGG_ROWS, 32), jnp.float32),
        scratch_types=[
            pltpu.VMEM((_NBR, _CH), jnp.int32),
            pltpu.VMEM((_NBR, _CH), jnp.int32),
            pltpu.VMEM((_NBR, _CH), jnp.int32),
            pltpu.VMEM((_CH, 32), jnp.float32),
            pltpu.VMEM((_CH, 32), jnp.float32),
            pltpu.VMEM((_CH, 32), jnp.float32),
            pltpu.VMEM((_CH, 32), jnp.float32),
            pltpu.VMEM((_EF, _CH), jnp.float32),
            pltpu.VMEM((_EF, _CH), jnp.float32),
            pltpu.VMEM((_EF, _CH), jnp.float32),
            pltpu.VMEM((_EF, _CH), jnp.float32),
            pltpu.VMEM((_EF, 32), jnp.float32),
            pltpu.VMEM((_NG, 32), jnp.float32),
            pltpu.VMEM_SHARED((_AGG_ROWS, 32), jnp.float32),
            pltpu.SemaphoreType.DMA,
            pltpu.SemaphoreType.DMA,
            pltpu.SemaphoreType.DMA,
            pltpu.SemaphoreType.DMA,
            pltpu.SemaphoreType.DMA,
            pltpu.SemaphoreType.DMA,
            pltpu.SemaphoreType.DMA,
            pltpu.SemaphoreType.DMA,
        ],
    )
    def k(src_hbm, dst_hbm, gt_hbm, z_hbm, ea_hbm, wh_hbm, gp_hbm, zz_hbm,
          out_hbm, srcb, dstb, gtb, zr0, zr1, zr2, zr3, ea0b, ea1b, ea2b,
          ea3b, wv, gv, aggsh, gs0, gs1, gs2, gs3, ss0, ss1, ss2, ss3):
        c = lax.axis_index("c")
        s = lax.axis_index("s")
        pltpu.sync_copy(zz_hbm, aggsh.at[pl.ds(s * _ZSL, _ZSL)])
        pltpu.sync_copy(wh_hbm.at[c], wv)
        pltpu.sync_copy(gp_hbm.at[c], gv)
        plsc.subcore_barrier()
        iota16 = lax.iota(jnp.int32, 16)
        zero16 = jnp.zeros((16,), jnp.int32)
        wvec = [(wv[kk, pl.ds(0, 16)], wv[kk, pl.ds(16, 16)])
                for kk in range(_EF)]
        zrs = [zr0, zr1, zr2, zr3]
        eas = [ea0b, ea1b, ea2b, ea3b]
        gsems = [gs0, gs1, gs2, gs3]
        ssems = [ss0, ss1, ss2, ss3]
        zoff = c * _N
        base = s * _RPT

        def drain(sem, buf):
            pltpu.make_async_copy(zz_hbm.at[pl.ds(0, _CH)], buf, sem).wait()

        def drain_ea(sem, buf):
            pltpu.make_async_copy(ea_hbm.at[:, pl.ds(0, _CH)], buf, sem).wait()

        def fire(slot, jj, row0):
            pltpu.async_copy(z_hbm.at[srcb.at[jj]], zrs[slot], gsems[slot])
            col0 = (row0 + jj) * _CH
            for kk in range(_EF):
                pltpu.async_copy(ea_hbm.at[kk, pl.ds(col0, _CH)],
                                 eas[slot].at[kk], gsems[slot])

        def compute(jj, zr, eab):
            jv = zero16 + jj

            @plsc.parallel_loop(0, _CH, unroll=4)
            def comp(e):
                ev = zero16 + e
                gt_b = plsc.load_gather(gtb, [jv, ev])
                ea0 = plsc.load_gather(eab, [zero16, ev])
                ea1 = plsc.load_gather(eab, [zero16 + 1, ev])
                ea2 = plsc.load_gather(eab, [zero16 + 2, ev])
                ea3 = plsc.load_gather(eab, [zero16 + 3, ev])
                g0 = plsc.load_gather(gv, [gt_b, iota16])
                g1 = plsc.load_gather(gv, [gt_b, iota16 + 16])
                acc0 = (g0 + wvec[0][0] * ea0 + wvec[1][0] * ea1
                        + wvec[2][0] * ea2 + wvec[3][0] * ea3)
                acc1 = (g1 + wvec[0][1] * ea0 + wvec[1][1] * ea1
                        + wvec[2][1] * ea2 + wvec[3][1] * ea3)
                s0 = pl.ds(0, 16)
                s1 = pl.ds(16, 16)
                zr[e, s0] = jnp.maximum(acc0 + zr[e, s0], 0.0)
                zr[e, s1] = jnp.maximum(acc1 + zr[e, s1], 0.0)

        def batch(b, _):
            row0 = base + b * _NBR
            pltpu.sync_copy(src_hbm.at[pl.ds(row0, _NBR)], srcb)
            pltpu.sync_copy(dst_hbm.at[pl.ds(row0, _NBR)], dstb)
            pltpu.sync_copy(gt_hbm.at[pl.ds(row0, _NBR)], gtb)

            def adj(r, _):
                for kk in range(_CH // 16):
                    sl = pl.ds(kk * 16, 16)
                    srcb[r, sl] = srcb[r, sl] + zoff
                return 0

            lax.fori_loop(0, _NBR, adj, 0)

            @pl.when(b > 0)
            def _():
                for r in range(4):
                    drain(ssems[r], zrs[r])

            for r in range(3):
                fire(r, r, row0)

            def grp(g, _):
                for r in range(4):
                    jj = g * 4 + r
                    drain(gsems[r], zrs[r])
                    drain_ea(gsems[r], eas[r])
                    compute(jj, zrs[r], eas[r])
                    pltpu.async_copy(zrs[r], aggsh.at[dstb.at[jj]],
                                     ssems[r], add=True)
                    nslot = (r + 3) % 4

                    @pl.when(jnp.logical_and(jj >= 1, jj < _NBR - 3))
                    def _():
                        drain(ssems[nslot], zrs[nslot])

                    @pl.when(jj < _NBR - 3)
                    def _():
                        fire(nslot, jj + 3, row0)

                return 0

            lax.fori_loop(0, _NBR // 4, grp, 0)
            return 0

        lax.fori_loop(0, _NBATCH, batch, 0)
        for r in range(4):
            drain(ssems[r], zrs[r])
        plsc.subcore_barrier()
        pltpu.sync_copy(aggsh.at[pl.ds(s * _ZSL, _ZSL)],
                        out_hbm.at[c, pl.ds(s * _ZSL, _ZSL)])

    return k(src2d, dst2d, gt2d, zcat, ea2, wh, gp, zeros32)


def _maxpool_kernel(hp, batchp):
    @functools.partial(
        pl.kernel, mesh=_MESH,
        compiler_params=pltpu.CompilerParams(use_tc_tiling_on_sc=False,
                                             needs_layout_passes=False),
        out_type=jax.ShapeDtypeStruct((2, _B, _H), jnp.float32),
        scratch_types=[
            pltpu.VMEM((_PHALF, _H), jnp.float32),
            pltpu.VMEM((_PHALF,), jnp.int32),
            pltpu.VMEM((_B, _H), jnp.float32),
            pltpu.VMEM((16, 8, _H), jnp.float32),
            pltpu.VMEM((8, _H), jnp.float32),
            pltpu.VMEM_SHARED((16, _B, _H), jnp.float32),
        ],
    )
    def k(hp_hbm, bp_hbm, out_hbm, hbuf, bbuf, maxb, buf16, outb, shmax):
        c = lax.axis_index("c")
        s = lax.axis_index("s")
        w = c * 16 + s
        iota16 = lax.iota(jnp.int32, 16)
        neg = jnp.full((16,), -1e30, jnp.float32)

        def ini(r, _):
            for cg in range(4):
                maxb[r, pl.ds(cg * 16, 16)] = neg
            return 0

        lax.fori_loop(0, _B, ini, 0)
        base = w * _PPT
        for half in range(2):
            pltpu.sync_copy(hp_hbm.at[pl.ds(base + half * _PHALF, _PHALF)], hbuf)
            pltpu.sync_copy(bp_hbm.at[pl.ds(base + half * _PHALF, _PHALF)], bbuf)

            def body(j, _):
                seg = plsc.load_gather(bbuf, [jnp.zeros((16,), jnp.int32) + j])
                for cg in range(4):
                    col = iota16 + cg * 16
                    cur = plsc.load_gather(maxb, [seg, col])
                    hv = hbuf[j, pl.ds(cg * 16, 16)]
                    plsc.store_scatter(maxb, [seg, col], jnp.maximum(cur, hv))
                return 0

            lax.fori_loop(0, _PHALF, body, 0)
        pltpu.sync_copy(maxb, shmax.at[s])
        plsc.subcore_barrier()

        @pl.when(s < 8)
        def _():
            for t2 in range(16):
                pltpu.sync_copy(shmax.at[t2, pl.ds(8 * s, 8)], buf16.at[t2])
            for si in range(8):
                for cg in range(4):
                    sl = pl.ds(cg * 16, 16)
                    acc = buf16[0, si, sl]
                    for t2 in range(1, 16):
                        acc = jnp.maximum(acc, buf16[t2, si, sl])
                    outb[si, sl] = acc
            pltpu.sync_copy(outb, out_hbm.at[c, pl.ds(8 * s, 8)])

    return k(hp, batchp)


def kernel(x, edge_index, edge_attr, edge_gate_type, batch, global_features,
           params):
    p = params
    src = edge_index[0]
    dst = edge_index[1]
    pad = _EPAD - _E
    src2d = jnp.concatenate([src, jnp.zeros((pad,), jnp.int32)]).reshape(_ROWS, _CH)
    dst2d = jnp.concatenate([dst, jnp.full((pad,), _N, jnp.int32)]).reshape(_ROWS, _CH)
    gt2d = jnp.concatenate([edge_gate_type,
                            jnp.zeros((pad,), jnp.int32)]).reshape(_ROWS, _CH)
    ea_p = jnp.concatenate([edge_attr.T,
                            jnp.zeros((_EF, pad), jnp.float32)], axis=1)

    gemb_s = jnp.stack([p['mp%d_gate_emb' % l] for l in range(_L)])
    wg_s = jnp.stack([p['mp%d_Wm' % l][_H + _EF:] for l in range(_L)])
    bm4 = jnp.stack([p['mp%d_bm' % l] for l in range(_L)])[:, None, :]
    bm4 = jnp.broadcast_to(bm4, (_L, _NG, _H))
    gp_all = _gproj(gemb_s, wg_s, bm4)
    whs = []
    gps = []
    for l in range(_L):
        we = p['mp%d_Wm' % l][_H:_H + _EF]
        whs.append(jnp.stack([we[:, :32], we[:, 32:]]))
        gps.append(jnp.stack([gp_all[l][:, :32], gp_all[l][:, 32:]]))

    zeros32 = jnp.zeros((_ZSL, 32), jnp.float32)
    zeros16 = jnp.zeros((_ZSL, 16), jnp.float32)
    ones16 = jnp.ones((_CH, 16), jnp.float32)

    h, zhalves = _embed_z(x, p['ne_W'], p['ne_b'][None, :], p['ne_ln_g'][None, :],
                          p['ne_ln_b'][None, :], p['mp0_Wm'][:_H])
    degp = _deg_kernel(dst2d, ones16, zeros16)

    for l in range(_L):
        zcat = zhalves.reshape(2 * _N, 32)
        agg = _edge_kernel(src2d, dst2d, gt2d, zcat, ea_p, whs[l], gps[l],
                           zeros32)
        if l < _L - 1:
            h, zhalves = _update(h, agg, degp, p['mp%d_Wu' % l],
                                 p['mp%d_bu' % l][None, :],
                                 wz=p['mp%d_Wm' % (l + 1)][:_H])
        else:
            h = _update(h, agg, degp, p['mp%d_Wu' % l],
                        p['mp%d_bu' % l][None, :])

    batchf = batch.astype(jnp.float32)[:, None]
    sums = _pool_sum(h, batchf)
    hp = jnp.concatenate([h, jnp.full((_NPOOL - _N, _H), -1e30, jnp.float32)])
    batchp = jnp.concatenate([batch, jnp.full((_NPOOL - _N,), _B - 1, jnp.int32)])
    maxp = _maxpool_kernel(hp, batchp)
    return _head(sums, maxp, global_features, p)
